# baseline (speedup 1.0000x reference)
import functools

import jax
import jax.numpy as jnp
from jax import lax
from jax.experimental import pallas as pl
from jax.experimental.pallas import tpu as pltpu
from jax.experimental.pallas import tpu_sc as plsc

_PH, _PW = 7, 7
_H, _W = 64, 64
_C = 512
_LANES = 16
_NC, _NS = 2, 16
_NW = _NC * _NS
_CPW = _C // _NW
_NCELL = _PH * _PW
_CHUNK = _CPW * _NCELL
_IDXPAD = 64


def _roi_pool_body(img_ref, rois_ref, out_ref,
                   roi_v, coord_ref, yi_ref, xi_ref, idx_v, rows_v, outb_v,
                   sem):
    wid = lax.axis_index("s") * _NC + lax.axis_index("c")
    lanes = lax.iota(jnp.int32, _LANES)

    pltpu.sync_copy(rois_ref.at[pl.ds(0, _LANES)], roi_v)
    coord_ref[...] = roi_v[...].astype(jnp.int32)

    def _lane(k):
        return plsc.load_gather(coord_ref, [jnp.full((_LANES,), k, jnp.int32)])

    x_min, y_min, x_max, y_max = _lane(1), _lane(2), _lane(3), _lane(4)
    h = y_max - y_min + 1
    w = x_max - x_min + 1

    yi_ref[...] = y_min + jnp.minimum((lanes * h) // _PH, h - 1)
    xi_ref[...] = x_min + jnp.minimum((lanes * w) // _PW, w - 1)

    for g in range(_IDXPAD // _LANES):
        k = jnp.minimum(g * _LANES + lanes, _NCELL - 1)
        i = k // _PW
        j = k - i * _PW
        yv = plsc.load_gather(yi_ref, [i])
        xv = plsc.load_gather(xi_ref, [j])
        idx_v[pl.ds(g * _LANES, _LANES)] = yv * _W + xv

    pltpu.async_copy(img_ref.at[idx_v], rows_v, sem).wait()

    cb = wid * _CPW
    for g in range(_CHUNK // _LANES):
        m = g * _LANES + lanes
        cl = m // _NCELL
        k = m - cl * _NCELL
        outb_v[pl.ds(g * _LANES, _LANES)] = plsc.load_gather(rows_v, [k, cb + cl])

    pltpu.sync_copy(outb_v, out_ref.at[pl.ds(wid * _CHUNK, _CHUNK)])


_roi_pool_sc = functools.partial(
    pl.kernel,
    out_type=jax.ShapeDtypeStruct((_NW * _CHUNK,), jnp.float32),
    mesh=plsc.VectorSubcoreMesh(core_axis_name="c", subcore_axis_name="s"),
    compiler_params=pltpu.CompilerParams(needs_layout_passes=False),
    scratch_types=[
        pltpu.VMEM((_LANES,), jnp.float32),
        pltpu.VMEM((_LANES,), jnp.int32),
        pltpu.VMEM((_LANES,), jnp.int32),
        pltpu.VMEM((_LANES,), jnp.int32),
        pltpu.VMEM((_IDXPAD,), jnp.int32),
        pltpu.VMEM((_IDXPAD, _C), jnp.float32),
        pltpu.VMEM((_CHUNK,), jnp.float32),
        pltpu.SemaphoreType.DMA,
    ],
)(_roi_pool_body)


def kernel(img, rois):
    img_rows = img.reshape(_H * _W, _C)
    out = _roi_pool_sc(img_rows, rois.reshape(-1))
    return out.reshape(_C, _PH, _PW)[None]


# speedup vs baseline: 1.5604x; 1.5604x over previous
import functools

import jax
import jax.numpy as jnp
from jax import lax
from jax.experimental import pallas as pl
from jax.experimental.pallas import tpu as pltpu
from jax.experimental.pallas import tpu_sc as plsc

_PH, _PW = 7, 7
_H, _W = 64, 64
_C = 512
_LANES = 16
_NC, _NS = 2, 16
_NW = _NC * _NS
_CPW = _C // _NW
_NCELL = _PH * _PW
_CHUNK = _CPW * _NCELL
_IDXPAD = 64
_ROW = 128


def _div7(x):
    return lax.shift_right_logical(x * 9363, 16)


def _div49(x):
    return lax.shift_right_logical(x * 1339, 16)


def _roi_pool_body(img_ref, rois_ref, out_ref,
                   roi_v, coord_ref, yi_ref, xi_ref, idx_v, rows_v, outb_v,
                   sem):
    wid = lax.axis_index("s") * _NC + lax.axis_index("c")
    lanes = lax.iota(jnp.int32, _LANES)

    pltpu.sync_copy(rois_ref.at[pl.ds(0, _LANES)], roi_v)
    coord_ref[...] = roi_v[...].astype(jnp.int32)

    def _lane(k):
        return plsc.load_gather(coord_ref, [jnp.full((_LANES,), k, jnp.int32)])

    x_min, y_min, x_max, y_max = _lane(1), _lane(2), _lane(3), _lane(4)
    h = y_max - y_min + 1
    w = x_max - x_min + 1

    yi_ref[...] = y_min + jnp.minimum(_div7(lanes * h), h - 1)
    xi_ref[...] = x_min + jnp.minimum(_div7(lanes * w), w - 1)

    for g in range(_IDXPAD // _LANES):
        k = jnp.minimum(g * _LANES + lanes, _NCELL - 1)
        i = _div7(k)
        j = k - i * _PW
        yv = plsc.load_gather(yi_ref, [i])
        xv = plsc.load_gather(xi_ref, [j])
        idx_v[pl.ds(g * _LANES, _LANES)] = yv * _W + xv

    q = wid // (_ROW // _CPW)
    cb = (wid % (_ROW // _CPW)) * _CPW
    pltpu.async_copy(img_ref.at[idx_v, pl.ds(q * _ROW, _ROW)], rows_v, sem).wait()

    for g in range(_CHUNK // _LANES):
        m = g * _LANES + lanes
        cl = _div49(m)
        k = m - cl * _NCELL
        outb_v[pl.ds(g * _LANES, _LANES)] = plsc.load_gather(rows_v, [k, cb + cl])

    pltpu.sync_copy(outb_v, out_ref.at[pl.ds(wid * _CHUNK, _CHUNK)])


_roi_pool_sc = functools.partial(
    pl.kernel,
    out_type=jax.ShapeDtypeStruct((_NW * _CHUNK,), jnp.float32),
    mesh=plsc.VectorSubcoreMesh(core_axis_name="c", subcore_axis_name="s"),
    compiler_params=pltpu.CompilerParams(needs_layout_passes=False),
    scratch_types=[
        pltpu.VMEM((_LANES,), jnp.float32),
        pltpu.VMEM((_LANES,), jnp.int32),
        pltpu.VMEM((_LANES,), jnp.int32),
        pltpu.VMEM((_LANES,), jnp.int32),
        pltpu.VMEM((_IDXPAD,), jnp.int32),
        pltpu.VMEM((_IDXPAD, _ROW), jnp.float32),
        pltpu.VMEM((_CHUNK,), jnp.float32),
        pltpu.SemaphoreType.DMA,
    ],
)(_roi_pool_body)


def kernel(img, rois):
    img_rows = img.reshape(_H * _W, _C)
    out = _roi_pool_sc(img_rows, rois.reshape(-1))
    return out.reshape(_C, _PH, _PW)[None]


# speedup vs baseline: 2.0663x; 1.3242x over previous
import functools

import jax
import jax.numpy as jnp
from jax import lax
from jax.experimental import pallas as pl
from jax.experimental.pallas import tpu as pltpu
from jax.experimental.pallas import tpu_sc as plsc

_PH, _PW = 7, 7
_H, _W = 64, 64
_C = 512
_LANES = 16
_NC, _NS = 2, 16
_NCELL = _PH * _PW


def _div7(x):
    return lax.shift_right_logical(x * 9363, 16)


def _roi_pool_body(img_ref, rois_ref, out_ref,
                   roi_v, yi_ref, xi_ref, idx_v, rows_v, sem):
    wid = lax.axis_index("s") * _NC + lax.axis_index("c")

    @pl.when(wid < _PH)
    def _():
        lanes = lax.iota(jnp.int32, _LANES)

        pltpu.sync_copy(rois_ref.at[0], roi_v)

        def _lane(k):
            v = plsc.load_gather(roi_v, [jnp.full((_LANES,), k, jnp.int32)])
            return v.astype(jnp.int32)

        x_min, y_min, x_max, y_max = _lane(1), _lane(2), _lane(3), _lane(4)
        h = y_max - y_min + 1
        w = x_max - x_min + 1

        yi = y_min + jnp.minimum(_div7(lanes * h), h - 1)
        xi = x_min + jnp.minimum(_div7(lanes * w), w - 1)
        yi_ref[...] = yi
        xi_ref[...] = xi

        yrow = plsc.load_gather(yi_ref, [jnp.full((_LANES,), wid, jnp.int32)])
        xcol = plsc.load_gather(xi_ref, [jnp.minimum(lanes, _PW - 1)])
        idx_v[...] = yrow * _W + xcol

        gathers = [
            pltpu.async_copy(
                img_ref.at[idx_v.at[pl.ds(0, 8)], pl.ds(qb * 128, 128)],
                rows_v.at[pl.ds(qb * 8, 8)], sem)
            for qb in range(_C // 128)
        ]
        for g in gathers:
            g.wait()
        stores = [
            pltpu.async_copy(
                rows_v.at[pl.ds(qb * 8, _PW)],
                out_ref.at[wid, :, pl.ds(qb * 128, 128)], sem)
            for qb in range(_C // 128)
        ]
        for s in stores:
            s.wait()


_roi_pool_sc = functools.partial(
    pl.kernel,
    out_type=jax.ShapeDtypeStruct((_PH, _PW, _C), jnp.float32),
    mesh=plsc.VectorSubcoreMesh(
        core_axis_name="c", subcore_axis_name="s", num_cores=_NC),
    compiler_params=pltpu.CompilerParams(needs_layout_passes=False),
    scratch_types=[
        pltpu.VMEM((5,), jnp.float32),
        pltpu.VMEM((_LANES,), jnp.int32),
        pltpu.VMEM((_LANES,), jnp.int32),
        pltpu.VMEM((_LANES,), jnp.int32),
        pltpu.VMEM((32, 128), jnp.float32),
        pltpu.SemaphoreType.DMA,
    ],
)(_roi_pool_body)


def kernel(img, rois):
    img_rows = img.reshape(_H * _W, _C)
    out = _roi_pool_sc(img_rows, rois)
    return jnp.transpose(out, (2, 0, 1))[None]


# speedup vs baseline: 2.1905x; 1.0601x over previous
import functools

import jax
import jax.numpy as jnp
from jax import lax
from jax.experimental import pallas as pl
from jax.experimental.pallas import tpu as pltpu
from jax.experimental.pallas import tpu_sc as plsc

_PH, _PW = 7, 7
_H, _W = 64, 64
_C = 512
_LANES = 16
_NC, _NS = 2, 16
_NCELL = _PH * _PW
_MESH_NC = 1


def _div7(x):
    return lax.shift_right_logical(x * 9363, 16)


def _roi_pool_body(img_ref, rois_ref, out_ref,
                   roi_v, yi_ref, xi_ref, idx_v, rows_v, sem):
    wid = lax.axis_index("s") * _MESH_NC + lax.axis_index("c")

    @pl.when(wid < 2 * _PH)
    def _():
        lanes = lax.iota(jnp.int32, _LANES)
        i = wid // 2
        ch = (wid - i * 2) * 256

        pltpu.sync_copy(rois_ref.at[pl.ds(0, 5), pl.ds(0, 128)], roi_v)

        def _lane(k):
            v = plsc.load_gather(
                roi_v, [jnp.full((_LANES,), k, jnp.int32),
                        jnp.zeros((_LANES,), jnp.int32)])
            return v.astype(jnp.int32)

        x_min, y_min, x_max, y_max = _lane(1), _lane(2), _lane(3), _lane(4)
        h = y_max - y_min + 1
        w = x_max - x_min + 1

        yi = y_min + jnp.minimum(_div7(lanes * h), h - 1)
        xi = x_min + jnp.minimum(_div7(lanes * w), w - 1)
        yi_ref[...] = yi
        xi_ref[...] = xi

        yrow = plsc.load_gather(yi_ref, [jnp.full((_LANES,), i, jnp.int32)])
        xcol = plsc.load_gather(xi_ref, [jnp.minimum(lanes, _PW - 1)])
        idx_v[...] = yrow * _W + xcol

        pltpu.async_copy(
            img_ref.at[idx_v.at[pl.ds(0, 8)], pl.ds(ch, 256)],
            rows_v, sem).wait()
        pltpu.sync_copy(
            rows_v.at[pl.ds(0, _PW)], out_ref.at[i, :, pl.ds(ch, 256)])


_roi_pool_sc = functools.partial(
    pl.kernel,
    out_type=jax.ShapeDtypeStruct((_PH, _PW, _C), jnp.float32),
    mesh=plsc.VectorSubcoreMesh(
        core_axis_name="c", subcore_axis_name="s", num_cores=_MESH_NC),
    compiler_params=pltpu.CompilerParams(needs_layout_passes=False),
    scratch_types=[
        pltpu.VMEM((5, 128), jnp.float32),
        pltpu.VMEM((_LANES,), jnp.int32),
        pltpu.VMEM((_LANES,), jnp.int32),
        pltpu.VMEM((_LANES,), jnp.int32),
        pltpu.VMEM((8, 256), jnp.float32),
        pltpu.SemaphoreType.DMA,
    ],
)(_roi_pool_body)


def kernel(img, rois):
    img_rows = img.reshape(_H * _W, _C)
    out = _roi_pool_sc(img_rows, rois.T)
    return jnp.transpose(out, (2, 0, 1))[None]
